# baseline (speedup 1.0000x reference)
import functools

import jax
import jax.numpy as jnp
from jax import lax
from jax.experimental import pallas as pl
from jax.experimental.pallas import tpu as pltpu
from jax.experimental.pallas import tpu_sc as plsc

VOCAB = 100000
EMBED_DIM = 128
BATCH = 4096
HIST = 200

N = BATCH * HIST
NC, NS = 2, 16
NW = NC * NS
PER_W = N // NW
CHUNK = 128
NCHUNK = PER_W // CHUNK
K = 2
GROUP = K * CHUNK
NPAIR = NCHUNK // (2 * K)

_mesh = plsc.VectorSubcoreMesh(core_axis_name="c", subcore_axis_name="s")


@functools.partial(
    pl.kernel,
    out_type=jax.ShapeDtypeStruct((N, EMBED_DIM), jnp.float32),
    mesh=_mesh,
    scratch_types=[
        pltpu.VMEM((NCHUNK, CHUNK), jnp.int32),
        pltpu.VMEM((2, GROUP, EMBED_DIM), jnp.float32),
        pltpu.SemaphoreType.DMA((2, K)),
        pltpu.SemaphoreType.DMA((2,)),
    ],
)
def _gather_kernel(x_hbm, table_hbm, out_hbm, idx_v, rings_v, gsem, ssem):
    wid = lax.axis_index("s") * NC + lax.axis_index("c")
    pltpu.sync_copy(x_hbm.at[pl.ds(wid * NCHUNK, NCHUNK)], idx_v)
    base = wid * PER_W

    def g_start(j, r, b):
        pltpu.async_copy(
            table_hbm.at[idx_v.at[j]],
            rings_v.at[r, pl.ds(b * CHUNK, CHUNK)],
            gsem.at[r, b],
        )

    def g_wait(r, b):
        pltpu.make_async_copy(
            table_hbm.at[idx_v.at[0]],
            rings_v.at[r, pl.ds(b * CHUNK, CHUNK)],
            gsem.at[r, b],
        ).wait()

    def s_start(g, r):
        pltpu.async_copy(
            rings_v.at[r], out_hbm.at[pl.ds(base + g * GROUP, GROUP)], ssem.at[r]
        )

    def s_wait(r):
        pltpu.make_async_copy(
            rings_v.at[r], out_hbm.at[pl.ds(0, GROUP)], ssem.at[r]
        ).wait()

    for r in range(2):
        for b in range(K):
            g_start(r * K + b, r, b)

    def body(t, carry):
        c0 = t * (2 * K)
        for b in range(K):
            g_wait(0, b)
        s_start(2 * t, 0)
        for b in range(K):
            g_wait(1, b)
        s_start(2 * t + 1, 1)
        s_wait(0)
        for b in range(K):
            g_start(c0 + 2 * K + b, 0, b)
        s_wait(1)
        for b in range(K):
            g_start(c0 + 3 * K + b, 1, b)
        return carry

    lax.fori_loop(0, NPAIR - 1, body, 0)

    for b in range(K):
        g_wait(0, b)
    s_start(2 * (NPAIR - 1), 0)
    for b in range(K):
        g_wait(1, b)
    s_start(2 * (NPAIR - 1) + 1, 1)
    s_wait(0)
    s_wait(1)


def kernel(x, table):
    xf = x.reshape(-1).astype(jnp.int32).reshape(N // CHUNK, CHUNK)
    out = _gather_kernel(xf, table)
    return out.reshape(BATCH, HIST, EMBED_DIM)


# speedup vs baseline: 1.0135x; 1.0135x over previous
import functools

import jax
import jax.numpy as jnp
from jax import lax
from jax.experimental import pallas as pl
from jax.experimental.pallas import tpu as pltpu
from jax.experimental.pallas import tpu_sc as plsc

VOCAB = 100000
EMBED_DIM = 128
BATCH = 4096
HIST = 200

N = BATCH * HIST
NC, NS = 2, 16
NW = NC * NS
PER_W = N // NW
CHUNK = 80
NCHUNK = PER_W // CHUNK
K = 4
GROUP = K * CHUNK
NPAIR = NCHUNK // (2 * K)

_mesh = plsc.VectorSubcoreMesh(core_axis_name="c", subcore_axis_name="s")


@functools.partial(
    pl.kernel,
    out_type=jax.ShapeDtypeStruct((N, EMBED_DIM), jnp.float32),
    mesh=_mesh,
    scratch_types=[
        pltpu.VMEM((NCHUNK, CHUNK), jnp.int32),
        pltpu.VMEM((2, GROUP, EMBED_DIM), jnp.float32),
        pltpu.SemaphoreType.DMA((2, K)),
        pltpu.SemaphoreType.DMA((2,)),
    ],
)
def _gather_kernel(x_hbm, table_hbm, out_hbm, idx_v, rings_v, gsem, ssem):
    wid = lax.axis_index("s") * NC + lax.axis_index("c")
    pltpu.sync_copy(x_hbm.at[pl.ds(wid * NCHUNK, NCHUNK)], idx_v)
    base = wid * PER_W

    def g_start(j, r, b):
        pltpu.async_copy(
            table_hbm.at[idx_v.at[j]],
            rings_v.at[r, pl.ds(b * CHUNK, CHUNK)],
            gsem.at[r, b],
        )

    def g_wait(r, b):
        pltpu.make_async_copy(
            table_hbm.at[idx_v.at[0]],
            rings_v.at[r, pl.ds(b * CHUNK, CHUNK)],
            gsem.at[r, b],
        ).wait()

    def s_start(g, r):
        pltpu.async_copy(
            rings_v.at[r], out_hbm.at[pl.ds(base + g * GROUP, GROUP)], ssem.at[r]
        )

    def s_wait(r):
        pltpu.make_async_copy(
            rings_v.at[r], out_hbm.at[pl.ds(0, GROUP)], ssem.at[r]
        ).wait()

    for r in range(2):
        for b in range(K):
            g_start(r * K + b, r, b)

    def body(t, carry):
        c0 = t * (2 * K)
        for b in range(K):
            g_wait(0, b)
        s_start(2 * t, 0)
        for b in range(K):
            g_wait(1, b)
        s_start(2 * t + 1, 1)
        s_wait(0)
        for b in range(K):
            g_start(c0 + 2 * K + b, 0, b)
        s_wait(1)
        for b in range(K):
            g_start(c0 + 3 * K + b, 1, b)
        return carry

    lax.fori_loop(0, NPAIR - 1, body, 0)

    for b in range(K):
        g_wait(0, b)
    s_start(2 * (NPAIR - 1), 0)
    for b in range(K):
        g_wait(1, b)
    s_start(2 * (NPAIR - 1) + 1, 1)
    s_wait(0)
    s_wait(1)


def kernel(x, table):
    xf = x.reshape(-1).astype(jnp.int32).reshape(N // CHUNK, CHUNK)
    out = _gather_kernel(xf, table)
    return out.reshape(BATCH, HIST, EMBED_DIM)
